# Initial kernel scaffold; baseline (speedup 1.0000x reference)
import jax
import jax.numpy as jnp
from jax.experimental import pallas as pl


def kernel(x_0, incidence_1, W_init, b_init, W1, b1, W2, b2):
    raise NotImplementedError("write your pallas kernel here")


# speedup vs baseline: 1.0779x; 1.0779x over previous
import jax
import jax.numpy as jnp
from jax.experimental import pallas as pl
from jax.experimental.pallas import tpu as pltpu

_BK = 200


def _dot(a, b, dims):
    return jax.lax.dot_general(a, b, (dims, ((), ())),
                               preferred_element_type=jnp.float32)


def _pass1_kernel(x0_ref, a_ref, wi_ref, bi_ref, x_ref, abf_ref, x1_ref):
    k = pl.program_id(0)
    x_blk = _dot(x0_ref[...], wi_ref[...], ((1,), (1,))) + bi_ref[0, :]
    x_ref[...] = x_blk
    abf = a_ref[...].astype(jnp.bfloat16)
    abf_ref[...] = abf

    @pl.when(k == 0)
    def _():
        x1_ref[...] = jnp.zeros_like(x1_ref)

    x1_ref[...] += _dot(abf, x_blk.astype(jnp.bfloat16), ((0,), (0,)))


def _pass2_kernel(abf_ref, x_ref, x1_ref, w1_ref, b1_ref, xo_ref, x12_ref):
    i = pl.program_id(0)
    a = abf_ref[...]
    m = _dot(a, x1_ref[...].astype(jnp.bfloat16), ((1,), (0,)))
    xn = _dot(x_ref[...] + m, w1_ref[...], ((1,), (1,))) + b1_ref[0, :]
    xn = jnp.maximum(xn, 0.0)
    xo_ref[...] = xn

    @pl.when(i == 0)
    def _():
        x12_ref[...] = jnp.zeros_like(x12_ref)

    x12_ref[...] += _dot(a, xn.astype(jnp.bfloat16), ((0,), (0,)))


def _pass3_kernel(abf_ref, x_ref, x12_ref, w2_ref, b2_ref, xo_ref):
    a = abf_ref[...]
    m = _dot(a, x12_ref[...].astype(jnp.bfloat16), ((1,), (0,)))
    xn = _dot(x_ref[...] + m, w2_ref[...], ((1,), (1,))) + b2_ref[0, :]
    xo_ref[...] = jnp.maximum(xn, 0.0)


def kernel(x_0, incidence_1, W_init, b_init, W1, b1, W2, b2):
    n, in_ch = x_0.shape
    hid = W_init.shape[0]
    n_edges = incidence_1.shape[1]
    steps = n // _BK

    bi = b_init.reshape(1, hid)
    b1r = b1.reshape(1, hid)
    b2r = b2.reshape(1, hid)

    x_l0, a_bf, x1_l1 = pl.pallas_call(
        _pass1_kernel,
        grid=(steps,),
        in_specs=[
            pl.BlockSpec((_BK, in_ch), lambda k: (k, 0)),
            pl.BlockSpec((_BK, n_edges), lambda k: (k, 0)),
            pl.BlockSpec((hid, in_ch), lambda k: (0, 0)),
            pl.BlockSpec((1, hid), lambda k: (0, 0)),
        ],
        out_specs=[
            pl.BlockSpec((_BK, hid), lambda k: (k, 0)),
            pl.BlockSpec((_BK, n_edges), lambda k: (k, 0)),
            pl.BlockSpec((n_edges, hid), lambda k: (0, 0)),
        ],
        out_shape=[
            jax.ShapeDtypeStruct((n, hid), jnp.float32),
            jax.ShapeDtypeStruct((n, n_edges), jnp.bfloat16),
            jax.ShapeDtypeStruct((n_edges, hid), jnp.float32),
        ],
    )(x_0, incidence_1, W_init, bi)

    x_l1, x1_l2 = pl.pallas_call(
        _pass2_kernel,
        grid=(steps,),
        in_specs=[
            pl.BlockSpec((_BK, n_edges), lambda i: (i, 0)),
            pl.BlockSpec((_BK, hid), lambda i: (i, 0)),
            pl.BlockSpec((n_edges, hid), lambda i: (0, 0)),
            pl.BlockSpec((hid, hid), lambda i: (0, 0)),
            pl.BlockSpec((1, hid), lambda i: (0, 0)),
        ],
        out_specs=[
            pl.BlockSpec((_BK, hid), lambda i: (i, 0)),
            pl.BlockSpec((n_edges, hid), lambda i: (0, 0)),
        ],
        out_shape=[
            jax.ShapeDtypeStruct((n, hid), jnp.float32),
            jax.ShapeDtypeStruct((n_edges, hid), jnp.float32),
        ],
    )(a_bf, x_l0, x1_l1, W1, b1r)

    x_out = pl.pallas_call(
        _pass3_kernel,
        grid=(steps,),
        in_specs=[
            pl.BlockSpec((_BK, n_edges), lambda i: (i, 0)),
            pl.BlockSpec((_BK, hid), lambda i: (i, 0)),
            pl.BlockSpec((n_edges, hid), lambda i: (0, 0)),
            pl.BlockSpec((hid, hid), lambda i: (0, 0)),
            pl.BlockSpec((1, hid), lambda i: (0, 0)),
        ],
        out_specs=pl.BlockSpec((_BK, hid), lambda i: (i, 0)),
        out_shape=jax.ShapeDtypeStruct((n, hid), jnp.float32),
    )(a_bf, x_l1, x1_l2, W2, b2r)

    return x_out, x1_l2


# speedup vs baseline: 1.1601x; 1.0763x over previous
import jax
import jax.numpy as jnp
from jax.experimental import pallas as pl
from jax.experimental.pallas import tpu as pltpu

_BK = 200


def _dot(a, b, dims):
    return jax.lax.dot_general(a, b, (dims, ((), ())),
                               preferred_element_type=jnp.float32)


def _pass1_kernel(x0_ref, a_ref, wit_ref, bi_ref, x_ref, abf_ref, x1t_ref):
    k = pl.program_id(0)
    x_blk = _dot(x0_ref[...], wit_ref[...], ((1,), (0,))) + bi_ref[0, :]
    x_ref[...] = x_blk
    abf = a_ref[...].astype(jnp.bfloat16)
    abf_ref[...] = abf

    @pl.when(k == 0)
    def _():
        x1t_ref[...] = jnp.zeros_like(x1t_ref)

    x1t_ref[...] += _dot(x_blk.astype(jnp.bfloat16), abf, ((0,), (0,)))


def _pass2_kernel(abf_ref, x_ref, x1_ref, w1t_ref, b1_ref, xo_ref, x12t_ref):
    i = pl.program_id(0)
    a = abf_ref[...]
    m = _dot(a, x1_ref[...], ((1,), (0,)))
    xn = _dot(x_ref[...] + m, w1t_ref[...], ((1,), (0,))) + b1_ref[0, :]
    xn = jnp.maximum(xn, 0.0)
    xo_ref[...] = xn

    @pl.when(i == 0)
    def _():
        x12t_ref[...] = jnp.zeros_like(x12t_ref)

    x12t_ref[...] += _dot(xn.astype(jnp.bfloat16), a, ((0,), (0,)))


def _pass3_kernel(abf_ref, x_ref, x12_ref, w2t_ref, b2_ref, xo_ref):
    a = abf_ref[...]
    m = _dot(a, x12_ref[...], ((1,), (0,)))
    xn = _dot(x_ref[...] + m, w2t_ref[...], ((1,), (0,))) + b2_ref[0, :]
    xo_ref[...] = jnp.maximum(xn, 0.0)


def kernel(x_0, incidence_1, W_init, b_init, W1, b1, W2, b2):
    n, in_ch = x_0.shape
    hid = W_init.shape[0]
    n_edges = incidence_1.shape[1]
    steps = n // _BK

    bi = b_init.reshape(1, hid)
    b1r = b1.reshape(1, hid)
    b2r = b2.reshape(1, hid)

    x_l0, a_bf, x1t = pl.pallas_call(
        _pass1_kernel,
        grid=(steps,),
        in_specs=[
            pl.BlockSpec((_BK, in_ch), lambda k: (k, 0)),
            pl.BlockSpec((_BK, n_edges), lambda k: (k, 0)),
            pl.BlockSpec((in_ch, hid), lambda k: (0, 0)),
            pl.BlockSpec((1, hid), lambda k: (0, 0)),
        ],
        out_specs=[
            pl.BlockSpec((_BK, hid), lambda k: (k, 0)),
            pl.BlockSpec((_BK, n_edges), lambda k: (k, 0)),
            pl.BlockSpec((hid, n_edges), lambda k: (0, 0)),
        ],
        out_shape=[
            jax.ShapeDtypeStruct((n, hid), jnp.float32),
            jax.ShapeDtypeStruct((n, n_edges), jnp.bfloat16),
            jax.ShapeDtypeStruct((hid, n_edges), jnp.float32),
        ],
    )(x_0, incidence_1, W_init.T, bi)

    x1_bf = x1t.T.astype(jnp.bfloat16)

    x_l1, x12t = pl.pallas_call(
        _pass2_kernel,
        grid=(steps,),
        in_specs=[
            pl.BlockSpec((_BK, n_edges), lambda i: (i, 0)),
            pl.BlockSpec((_BK, hid), lambda i: (i, 0)),
            pl.BlockSpec((n_edges, hid), lambda i: (0, 0)),
            pl.BlockSpec((hid, hid), lambda i: (0, 0)),
            pl.BlockSpec((1, hid), lambda i: (0, 0)),
        ],
        out_specs=[
            pl.BlockSpec((_BK, hid), lambda i: (i, 0)),
            pl.BlockSpec((hid, n_edges), lambda i: (0, 0)),
        ],
        out_shape=[
            jax.ShapeDtypeStruct((n, hid), jnp.float32),
            jax.ShapeDtypeStruct((hid, n_edges), jnp.float32),
        ],
    )(a_bf, x_l0, x1_bf, W1.T, b1r)

    x1_l2 = x12t.T
    x12_bf = x1_l2.astype(jnp.bfloat16)

    x_out = pl.pallas_call(
        _pass3_kernel,
        grid=(steps,),
        in_specs=[
            pl.BlockSpec((_BK, n_edges), lambda i: (i, 0)),
            pl.BlockSpec((_BK, hid), lambda i: (i, 0)),
            pl.BlockSpec((n_edges, hid), lambda i: (0, 0)),
            pl.BlockSpec((hid, hid), lambda i: (0, 0)),
            pl.BlockSpec((1, hid), lambda i: (0, 0)),
        ],
        out_specs=pl.BlockSpec((_BK, hid), lambda i: (i, 0)),
        out_shape=jax.ShapeDtypeStruct((n, hid), jnp.float32),
    )(a_bf, x_l1, x12_bf, W2.T, b2r)

    return x_out, x1_l2


# speedup vs baseline: 1.2535x; 1.0805x over previous
import jax
import jax.numpy as jnp
from jax.experimental import pallas as pl
from jax.experimental.pallas import tpu as pltpu

_BK = 200
_BK23 = 400


def _dot(a, b, dims):
    return jax.lax.dot_general(a, b, (dims, ((), ())),
                               preferred_element_type=jnp.float32)


def _pass1_kernel(x0_ref, a_ref, wit_ref, bi_ref, x_ref, abf_ref, x1t_ref):
    k = pl.program_id(0)
    x_blk = _dot(x0_ref[...], wit_ref[...], ((1,), (0,))) + bi_ref[0, :]
    x_ref[...] = x_blk
    abf = a_ref[...].astype(jnp.bfloat16)
    abf_ref[...] = abf

    @pl.when(k == 0)
    def _():
        x1t_ref[...] = jnp.zeros_like(x1t_ref)

    x1t_ref[...] += _dot(x_blk.astype(jnp.bfloat16), abf, ((0,), (0,)))


def _pass2_kernel(abf_ref, x_ref, x1_ref, w1t_ref, b1_ref, xo_ref, x12t_ref):
    i = pl.program_id(0)
    a = abf_ref[...]
    m = _dot(a, x1_ref[...], ((1,), (0,)))
    xn = _dot(x_ref[...] + m, w1t_ref[...], ((1,), (0,))) + b1_ref[0, :]
    xn = jnp.maximum(xn, 0.0)
    xo_ref[...] = xn

    @pl.when(i == 0)
    def _():
        x12t_ref[...] = jnp.zeros_like(x12t_ref)

    x12t_ref[...] += _dot(xn.astype(jnp.bfloat16), a, ((0,), (0,)))


def _pass3_kernel(abf_ref, x_ref, x12_ref, w2t_ref, b2_ref, xo_ref):
    a = abf_ref[...]
    m = _dot(a, x12_ref[...], ((1,), (0,)))
    xn = _dot(x_ref[...] + m, w2t_ref[...], ((1,), (0,))) + b2_ref[0, :]
    xo_ref[...] = jnp.maximum(xn, 0.0)


def kernel(x_0, incidence_1, W_init, b_init, W1, b1, W2, b2):
    n, in_ch = x_0.shape
    hid = W_init.shape[0]
    n_edges = incidence_1.shape[1]
    steps = n // _BK
    steps23 = n // _BK23

    bi = b_init.reshape(1, hid)
    b1r = b1.reshape(1, hid)
    b2r = b2.reshape(1, hid)

    x_l0, a_bf, x1t = pl.pallas_call(
        _pass1_kernel,
        grid=(steps,),
        in_specs=[
            pl.BlockSpec((_BK, in_ch), lambda k: (k, 0)),
            pl.BlockSpec((_BK, n_edges), lambda k: (k, 0)),
            pl.BlockSpec((in_ch, hid), lambda k: (0, 0)),
            pl.BlockSpec((1, hid), lambda k: (0, 0)),
        ],
        out_specs=[
            pl.BlockSpec((_BK, hid), lambda k: (k, 0)),
            pl.BlockSpec((_BK, n_edges), lambda k: (k, 0)),
            pl.BlockSpec((hid, n_edges), lambda k: (0, 0)),
        ],
        out_shape=[
            jax.ShapeDtypeStruct((n, hid), jnp.float32),
            jax.ShapeDtypeStruct((n, n_edges), jnp.bfloat16),
            jax.ShapeDtypeStruct((hid, n_edges), jnp.float32),
        ],
    )(x_0, incidence_1, W_init.T, bi)

    x1_bf = x1t.T.astype(jnp.bfloat16)

    x_l1, x12t = pl.pallas_call(
        _pass2_kernel,
        grid=(steps23,),
        in_specs=[
            pl.BlockSpec((_BK23, n_edges), lambda i: (i, 0)),
            pl.BlockSpec((_BK23, hid), lambda i: (i, 0)),
            pl.BlockSpec((n_edges, hid), lambda i: (0, 0)),
            pl.BlockSpec((hid, hid), lambda i: (0, 0)),
            pl.BlockSpec((1, hid), lambda i: (0, 0)),
        ],
        out_specs=[
            pl.BlockSpec((_BK23, hid), lambda i: (i, 0)),
            pl.BlockSpec((hid, n_edges), lambda i: (0, 0)),
        ],
        out_shape=[
            jax.ShapeDtypeStruct((n, hid), jnp.float32),
            jax.ShapeDtypeStruct((hid, n_edges), jnp.float32),
        ],
    )(a_bf, x_l0, x1_bf, W1.T, b1r)

    x1_l2 = x12t.T
    x12_bf = x1_l2.astype(jnp.bfloat16)

    x_out = pl.pallas_call(
        _pass3_kernel,
        grid=(steps23,),
        in_specs=[
            pl.BlockSpec((_BK23, n_edges), lambda i: (i, 0)),
            pl.BlockSpec((_BK23, hid), lambda i: (i, 0)),
            pl.BlockSpec((n_edges, hid), lambda i: (0, 0)),
            pl.BlockSpec((hid, hid), lambda i: (0, 0)),
            pl.BlockSpec((1, hid), lambda i: (0, 0)),
        ],
        out_specs=pl.BlockSpec((_BK23, hid), lambda i: (i, 0)),
        out_shape=jax.ShapeDtypeStruct((n, hid), jnp.float32),
    )(a_bf, x_l1, x12_bf, W2.T, b2r)

    return x_out, x1_l2


# speedup vs baseline: 1.2536x; 1.0001x over previous
import jax
import jax.numpy as jnp
from jax.experimental import pallas as pl
from jax.experimental.pallas import tpu as pltpu

_BK = 400
_BK23 = 400


def _dot(a, b, dims):
    return jax.lax.dot_general(a, b, (dims, ((), ())),
                               preferred_element_type=jnp.float32)


def _pass1_kernel(x0_ref, a_ref, wit_ref, bi_ref, x_ref, abf_ref, x1t_ref):
    k = pl.program_id(0)
    x_blk = _dot(x0_ref[...], wit_ref[...], ((1,), (0,))) + bi_ref[0, :]
    x_ref[...] = x_blk
    abf_ref[...] = a_ref[...].astype(jnp.bfloat16)

    @pl.when(k == 0)
    def _():
        x1t_ref[...] = jnp.zeros_like(x1t_ref)

    x1t_ref[...] += _dot(x_blk.astype(jnp.bfloat16), abf_ref[...], ((0,), (0,)))


def _pass2_kernel(abf_ref, x_ref, x1_ref, w1t_ref, b1_ref, xo_ref, x12t_ref):
    i = pl.program_id(0)
    m = _dot(abf_ref[...], x1_ref[...], ((1,), (0,)))
    xn = _dot(x_ref[...] + m, w1t_ref[...], ((1,), (0,))) + b1_ref[0, :]
    xn = jnp.maximum(xn, 0.0)
    xo_ref[...] = xn

    @pl.when(i == 0)
    def _():
        x12t_ref[...] = jnp.zeros_like(x12t_ref)

    x12t_ref[...] += _dot(xn.astype(jnp.bfloat16), abf_ref[...], ((0,), (0,)))


def _pass3_kernel(abf_ref, x_ref, x12_ref, w2t_ref, b2_ref, xo_ref):
    m = _dot(abf_ref[...], x12_ref[...], ((1,), (0,)))
    xn = _dot(x_ref[...] + m, w2t_ref[...], ((1,), (0,))) + b2_ref[0, :]
    xo_ref[...] = jnp.maximum(xn, 0.0)


def kernel(x_0, incidence_1, W_init, b_init, W1, b1, W2, b2):
    n, in_ch = x_0.shape
    hid = W_init.shape[0]
    n_edges = incidence_1.shape[1]
    steps = n // _BK
    steps23 = n // _BK23

    bi = b_init.reshape(1, hid)
    b1r = b1.reshape(1, hid)
    b2r = b2.reshape(1, hid)

    x_l0, a_bf, x1t = pl.pallas_call(
        _pass1_kernel,
        grid=(steps,),
        in_specs=[
            pl.BlockSpec((_BK, in_ch), lambda k: (k, 0)),
            pl.BlockSpec((_BK, n_edges), lambda k: (k, 0)),
            pl.BlockSpec((in_ch, hid), lambda k: (0, 0)),
            pl.BlockSpec((1, hid), lambda k: (0, 0)),
        ],
        out_specs=[
            pl.BlockSpec((_BK, hid), lambda k: (k, 0)),
            pl.BlockSpec((_BK, n_edges), lambda k: (k, 0)),
            pl.BlockSpec((hid, n_edges), lambda k: (0, 0)),
        ],
        out_shape=[
            jax.ShapeDtypeStruct((n, hid), jnp.float32),
            jax.ShapeDtypeStruct((n, n_edges), jnp.bfloat16),
            jax.ShapeDtypeStruct((hid, n_edges), jnp.float32),
        ],
    )(x_0, incidence_1, W_init.T, bi)

    x1_bf = x1t.T.astype(jnp.bfloat16)

    x_l1, x12t = pl.pallas_call(
        _pass2_kernel,
        grid=(steps23,),
        in_specs=[
            pl.BlockSpec((_BK23, n_edges), lambda i: (i, 0)),
            pl.BlockSpec((_BK23, hid), lambda i: (i, 0)),
            pl.BlockSpec((n_edges, hid), lambda i: (0, 0)),
            pl.BlockSpec((hid, hid), lambda i: (0, 0)),
            pl.BlockSpec((1, hid), lambda i: (0, 0)),
        ],
        out_specs=[
            pl.BlockSpec((_BK23, hid), lambda i: (i, 0)),
            pl.BlockSpec((hid, n_edges), lambda i: (0, 0)),
        ],
        out_shape=[
            jax.ShapeDtypeStruct((n, hid), jnp.float32),
            jax.ShapeDtypeStruct((hid, n_edges), jnp.float32),
        ],
    )(a_bf, x_l0, x1_bf, W1.T, b1r)

    x1_l2 = x12t.T
    x12_bf = x1_l2.astype(jnp.bfloat16)

    x_out = pl.pallas_call(
        _pass3_kernel,
        grid=(steps23,),
        in_specs=[
            pl.BlockSpec((_BK23, n_edges), lambda i: (i, 0)),
            pl.BlockSpec((_BK23, hid), lambda i: (i, 0)),
            pl.BlockSpec((n_edges, hid), lambda i: (0, 0)),
            pl.BlockSpec((hid, hid), lambda i: (0, 0)),
            pl.BlockSpec((1, hid), lambda i: (0, 0)),
        ],
        out_specs=pl.BlockSpec((_BK23, hid), lambda i: (i, 0)),
        out_shape=jax.ShapeDtypeStruct((n, hid), jnp.float32),
    )(a_bf, x_l1, x12_bf, W2.T, b2r)

    return x_out, x1_l2
